# Initial kernel scaffold; baseline (speedup 1.0000x reference)
import jax
import jax.numpy as jnp
from jax.experimental import pallas as pl


def kernel(x, edge_index, w1, b1, w2, b2):
    raise NotImplementedError("write your pallas kernel here")


# speedup vs baseline: 2.1481x; 2.1481x over previous
import jax
import jax.numpy as jnp
from jax.experimental import pallas as pl
from jax.experimental.pallas import tpu as pltpu


def _xw1_kernel(x_ref, w1_ref, s_ref, o_ref):
    xb = x_ref[...].astype(jnp.bfloat16)
    z = jnp.dot(xb, w1_ref[...], preferred_element_type=jnp.float32)
    o_ref[...] = (z * s_ref[...]).astype(jnp.bfloat16)


def _agg1_kernel(a_ref, xw1_ref, b1_ref, w2_ref, s_ref, o_ref):
    st = s_ref[...]
    z1 = jnp.dot(a_ref[...], xw1_ref[...], preferred_element_type=jnp.float32)
    h1 = jnp.maximum(z1 * st + b1_ref[...], 0.0).astype(jnp.bfloat16)
    g = jnp.dot(h1, w2_ref[...], preferred_element_type=jnp.float32)
    o_ref[...] = (g * st).astype(jnp.bfloat16)


def _agg2_kernel(a_ref, g_ref, b2_ref, s_ref, o_ref):
    z = jnp.dot(a_ref[...], g_ref[...], preferred_element_type=jnp.float32)
    z = z * s_ref[...] + b2_ref[...]
    m = jnp.max(z, axis=1, keepdims=True)
    zs = z - m
    lse = jnp.log(jnp.sum(jnp.exp(zs), axis=1, keepdims=True))
    o_ref[...] = zs - lse


def _round_up(v, m):
    return ((v + m - 1) // m) * m


def kernel(x, edge_index, w1, b1, w2, b2):
    n, f_in = x.shape
    h = w1.shape[1]
    c = w2.shape[1]

    tm = 256 if n >= 512 else 128
    n_pad = _round_up(n, tm)

    src, dst = edge_index
    diag = jnp.arange(n)
    one = jnp.ones((), jnp.bfloat16)
    adj = (jnp.zeros((n_pad, n_pad), jnp.bfloat16)
           .at[src, dst].set(one)
           .at[diag, diag].set(one))
    deg = jnp.sum(adj, axis=1, dtype=jnp.float32)
    s = jnp.where(deg > 0, jax.lax.rsqrt(jnp.maximum(deg, 1.0)), 0.0)
    s2 = s.reshape(n_pad, 1)

    if n_pad != n:
        x = jnp.zeros((n_pad, f_in), x.dtype).at[:n].set(x)

    w1_bf = w1.astype(jnp.bfloat16)
    w2_bf = w2.astype(jnp.bfloat16)
    b1_f = b1.astype(jnp.float32)
    b2_f = b2.astype(jnp.float32)

    grid = (n_pad // tm,)
    cparams = pltpu.CompilerParams(
        dimension_semantics=("parallel",), vmem_limit_bytes=64 * 2**20)

    xw1p = pl.pallas_call(
        _xw1_kernel,
        out_shape=jax.ShapeDtypeStruct((n_pad, h), jnp.bfloat16),
        grid=grid,
        in_specs=[
            pl.BlockSpec((tm, f_in), lambda i: (i, 0)),
            pl.BlockSpec((f_in, h), lambda i: (0, 0)),
            pl.BlockSpec((tm, 1), lambda i: (i, 0)),
        ],
        out_specs=pl.BlockSpec((tm, h), lambda i: (i, 0)),
        compiler_params=cparams,
    )(x, w1_bf, s2)

    g = pl.pallas_call(
        _agg1_kernel,
        out_shape=jax.ShapeDtypeStruct((n_pad, c), jnp.bfloat16),
        grid=grid,
        in_specs=[
            pl.BlockSpec((tm, n_pad), lambda i: (i, 0)),
            pl.BlockSpec((n_pad, h), lambda i: (0, 0)),
            pl.BlockSpec((1, h), lambda i: (0, 0)),
            pl.BlockSpec((h, c), lambda i: (0, 0)),
            pl.BlockSpec((tm, 1), lambda i: (i, 0)),
        ],
        out_specs=pl.BlockSpec((tm, c), lambda i: (i, 0)),
        compiler_params=cparams,
    )(adj, xw1p, b1_f, w2_bf, s2)

    out = pl.pallas_call(
        _agg2_kernel,
        out_shape=jax.ShapeDtypeStruct((n_pad, c), jnp.float32),
        grid=grid,
        in_specs=[
            pl.BlockSpec((tm, n_pad), lambda i: (i, 0)),
            pl.BlockSpec((n_pad, c), lambda i: (0, 0)),
            pl.BlockSpec((1, c), lambda i: (0, 0)),
            pl.BlockSpec((tm, 1), lambda i: (i, 0)),
        ],
        out_specs=pl.BlockSpec((tm, c), lambda i: (i, 0)),
        compiler_params=cparams,
    )(adj, g, b2_f, s2)

    return out[:n]


# speedup vs baseline: 8.2490x; 3.8401x over previous
import jax
import jax.numpy as jnp
from jax.experimental import pallas as pl
from jax.experimental.pallas import tpu as pltpu

_CK = 512
_SENTINEL = 1 << 20


def _build_kernel(starts_ref, src_ref, dst_ref, adj_ref, s_ref, acc_ref):
    t = pl.program_id(0)
    tm = adj_ref.shape[0]
    n_pad = adj_ref.shape[1]
    rows = t * tm + jax.lax.broadcasted_iota(jnp.int32, (tm, 1), 0)
    cols = jax.lax.broadcasted_iota(jnp.int32, (n_pad, 1), 0)

    acc_ref[...] = jnp.zeros_like(acc_ref)
    start = starts_ref[t]
    end = starts_ref[t + 1]
    base = (start // _CK) * _CK
    nch = (end - base + _CK - 1) // _CK

    def body(k, carry):
        off = pl.multiple_of(base + k * _CK, _CK)
        sl_src = src_ref[:, pl.ds(off, _CK)]
        sl_dst = dst_ref[:, pl.ds(off, _CK)]
        oh_src = (rows == sl_src).astype(jnp.bfloat16)
        oh_dst_t = (cols == sl_dst).astype(jnp.bfloat16)
        acc_ref[...] += jax.lax.dot_general(
            oh_src, oh_dst_t,
            dimension_numbers=(((1,), (1,)), ((), ())),
            preferred_element_type=jnp.float32)
        return carry

    jax.lax.fori_loop(0, nch, body, 0)

    eye = (rows == cols.reshape(1, n_pad)).astype(jnp.float32)
    adj = jnp.maximum(jnp.minimum(acc_ref[...], 1.0), eye)
    adj_ref[...] = adj.astype(jnp.bfloat16)
    deg = jnp.sum(adj, axis=1, keepdims=True)
    s_ref[...] = jax.lax.rsqrt(deg)


def _xw1_kernel(x_ref, w1_ref, s_ref, o_ref):
    xb = x_ref[...].astype(jnp.bfloat16)
    z = jnp.dot(xb, w1_ref[...], preferred_element_type=jnp.float32)
    o_ref[...] = (z * s_ref[...]).astype(jnp.bfloat16)


def _agg1_kernel(a_ref, xw1_ref, b1_ref, w2_ref, s_ref, o_ref):
    st = s_ref[...]
    z1 = jnp.dot(a_ref[...], xw1_ref[...], preferred_element_type=jnp.float32)
    h1 = jnp.maximum(z1 * st + b1_ref[...], 0.0).astype(jnp.bfloat16)
    g = jnp.dot(h1, w2_ref[...], preferred_element_type=jnp.float32)
    o_ref[...] = (g * st).astype(jnp.bfloat16)


def _agg2_kernel(a_ref, g_ref, b2_ref, s_ref, o_ref):
    z = jnp.dot(a_ref[...], g_ref[...], preferred_element_type=jnp.float32)
    z = z * s_ref[...] + b2_ref[...]
    m = jnp.max(z, axis=1, keepdims=True)
    zs = z - m
    lse = jnp.log(jnp.sum(jnp.exp(zs), axis=1, keepdims=True))
    o_ref[...] = zs - lse


def _round_up(v, m):
    return ((v + m - 1) // m) * m


def kernel(x, edge_index, w1, b1, w2, b2):
    n, f_in = x.shape
    h = w1.shape[1]
    c = w2.shape[1]

    tm = 256 if n >= 512 else 128
    n_pad = _round_up(n, tm)
    nt = n_pad // tm

    src = edge_index[0].astype(jnp.int32)
    dst = edge_index[1].astype(jnp.int32)
    e = src.shape[0]
    e_pad = _round_up(e, _CK)
    src_s, dst_s = jax.lax.sort_key_val(src, dst)
    starts = jnp.searchsorted(
        src_s, jnp.arange(nt + 1, dtype=jnp.int32) * tm).astype(jnp.int32)
    pad = jnp.full((e_pad - e,), _SENTINEL, jnp.int32)
    src_p = jnp.concatenate([src_s, pad]).reshape(1, e_pad)
    dst_p = jnp.concatenate([dst_s, jnp.zeros((e_pad - e,), jnp.int32)]
                            ).reshape(1, e_pad)

    if n_pad != n:
        x = jnp.zeros((n_pad, f_in), x.dtype).at[:n].set(x)

    w1_bf = w1.astype(jnp.bfloat16)
    w2_bf = w2.astype(jnp.bfloat16)
    b1_f = b1.astype(jnp.float32)
    b2_f = b2.astype(jnp.float32)

    grid = (nt,)
    cparams = pltpu.CompilerParams(
        dimension_semantics=("parallel",), vmem_limit_bytes=64 * 2**20)

    adj, s2 = pl.pallas_call(
        _build_kernel,
        grid_spec=pltpu.PrefetchScalarGridSpec(
            num_scalar_prefetch=1,
            grid=grid,
            in_specs=[
                pl.BlockSpec((1, e_pad), lambda i, st: (0, 0)),
                pl.BlockSpec((1, e_pad), lambda i, st: (0, 0)),
            ],
            out_specs=[
                pl.BlockSpec((tm, n_pad), lambda i, st: (i, 0)),
                pl.BlockSpec((tm, 1), lambda i, st: (i, 0)),
            ],
            scratch_shapes=[pltpu.VMEM((tm, n_pad), jnp.float32)],
        ),
        out_shape=[
            jax.ShapeDtypeStruct((n_pad, n_pad), jnp.bfloat16),
            jax.ShapeDtypeStruct((n_pad, 1), jnp.float32),
        ],
        compiler_params=cparams,
    )(starts, src_p, dst_p)

    xw1p = pl.pallas_call(
        _xw1_kernel,
        out_shape=jax.ShapeDtypeStruct((n_pad, h), jnp.bfloat16),
        grid=grid,
        in_specs=[
            pl.BlockSpec((tm, f_in), lambda i: (i, 0)),
            pl.BlockSpec((f_in, h), lambda i: (0, 0)),
            pl.BlockSpec((tm, 1), lambda i: (i, 0)),
        ],
        out_specs=pl.BlockSpec((tm, h), lambda i: (i, 0)),
        compiler_params=cparams,
    )(x, w1_bf, s2)

    g = pl.pallas_call(
        _agg1_kernel,
        out_shape=jax.ShapeDtypeStruct((n_pad, c), jnp.bfloat16),
        grid=grid,
        in_specs=[
            pl.BlockSpec((tm, n_pad), lambda i: (i, 0)),
            pl.BlockSpec((n_pad, h), lambda i: (0, 0)),
            pl.BlockSpec((1, h), lambda i: (0, 0)),
            pl.BlockSpec((h, c), lambda i: (0, 0)),
            pl.BlockSpec((tm, 1), lambda i: (i, 0)),
        ],
        out_specs=pl.BlockSpec((tm, c), lambda i: (i, 0)),
        compiler_params=cparams,
    )(adj, xw1p, b1_f, w2_bf, s2)

    out = pl.pallas_call(
        _agg2_kernel,
        out_shape=jax.ShapeDtypeStruct((n_pad, c), jnp.float32),
        grid=grid,
        in_specs=[
            pl.BlockSpec((tm, n_pad), lambda i: (i, 0)),
            pl.BlockSpec((n_pad, c), lambda i: (0, 0)),
            pl.BlockSpec((1, c), lambda i: (0, 0)),
            pl.BlockSpec((tm, 1), lambda i: (i, 0)),
        ],
        out_specs=pl.BlockSpec((tm, c), lambda i: (i, 0)),
        compiler_params=cparams,
    )(adj, g, b2_f, s2)

    return out[:n]
